# baseline (speedup 1.0000x reference)
import functools

import jax
import jax.numpy as jnp
from jax import lax
from jax.experimental import pallas as pl
from jax.experimental.pallas import tpu as pltpu
from jax.experimental.pallas import tpu_sc as plsc

N = 10000
E = 320000
NPAD = 10240
DEGW = 8
NC, NS = 2, 16
NW = NC * NS
CHUNK = 128
NBUF = 3
LAG = 1
K1 = 81
NCHUNK = E // CHUNK
TCHUNK = K1 * NW

_mesh = plsc.VectorSubcoreMesh(
    core_axis_name="c", subcore_axis_name="s", num_cores=NC, num_subcores=NS)
_sc_params = pltpu.CompilerParams(use_tc_tiling_on_sc=False)


def _deg_body(se_hbm, zeros_hbm, ones_hbm, out_hbm, idx_v, ones_v, acc_sh, sem):
    cid = lax.axis_index("c")
    sid = lax.axis_index("s")
    wid = cid * NS + sid
    pltpu.sync_copy(se_hbm.at[pl.ds(wid * K1, K1)], idx_v)
    pltpu.sync_copy(ones_hbm, ones_v)

    @pl.when(sid == 0)
    def _zero():
        pltpu.sync_copy(zeros_hbm, acc_sh)

    plsc.subcore_barrier()

    def body(j, carry):
        pltpu.async_copy(ones_v, acc_sh.at[idx_v.at[j, 1]], sem, add=True)
        return carry

    lax.fori_loop(0, K1, body, 0)

    def drain(j, carry):
        pltpu.make_async_copy(ones_v, acc_sh.at[idx_v.at[0, 1]], sem).wait()
        return carry

    lax.fori_loop(0, K1, drain, 0)
    plsc.subcore_barrier()

    @pl.when(sid == 0)
    def _flush():
        pltpu.sync_copy(acc_sh, out_hbm.at[cid])


_deg_kernel = functools.partial(
    pl.kernel,
    out_type=jax.ShapeDtypeStruct((NC, NPAD, DEGW), jnp.float32),
    mesh=_mesh,
    compiler_params=_sc_params,
    scratch_types=[
        pltpu.VMEM((K1, 2, CHUNK), jnp.int32),
        pltpu.VMEM((CHUNK, DEGW), jnp.float32),
        pltpu.VMEM_SHARED((NPAD, DEGW), jnp.float32),
        pltpu.SemaphoreType.DMA,
    ],
)(_deg_body)


def _make_scatter(D):
    def body(se_hbm, y_hbm, zeros_hbm, out_hbm,
             se_v, rows_v, acc_sh, y_sh, *sems):
        gsems, ssems = sems[:NBUF], sems[NBUF:]
        cid = lax.axis_index("c")
        sid = lax.axis_index("s")
        wid = cid * NS + sid
        pltpu.sync_copy(se_hbm.at[pl.ds(wid * K1, K1)], se_v)

        @pl.when(jnp.logical_and(sid == 0, cid == 0))
        def _init0():
            pltpu.sync_copy(y_hbm, acc_sh)

        @pl.when(jnp.logical_and(sid == 0, cid == 1))
        def _init1():
            pltpu.sync_copy(zeros_hbm, acc_sh)

        @pl.when(sid == 1)
        def _stage():
            pltpu.sync_copy(y_hbm, y_sh)

        plsc.subcore_barrier()
        for g in range(LAG):
            pltpu.async_copy(y_sh.at[se_v.at[g, 0]], rows_v.at[g], gsems[g])

        def step(g, carry):
            for b in range(NBUF):
                j = g * NBUF + b
                pltpu.make_async_copy(
                    y_sh.at[se_v.at[j, 0]], rows_v.at[b], gsems[b]).wait()
                pltpu.async_copy(
                    rows_v.at[b], acc_sh.at[se_v.at[j, 1]], ssems[b], add=True)
                b2 = (b + LAG) % NBUF

                @pl.when(j + LAG < K1)
                def _refill():
                    @pl.when(j >= NBUF - LAG)
                    def _wait_prev_scatter():
                        pltpu.make_async_copy(
                            rows_v.at[b2], acc_sh.at[se_v.at[0, 1]],
                            ssems[b2]).wait()

                    pltpu.async_copy(
                        y_sh.at[se_v.at[j + LAG, 0]], rows_v.at[b2], gsems[b2])
            return carry

        lax.fori_loop(0, K1 // NBUF, step, 0)
        for b in range(NBUF):
            pltpu.make_async_copy(
                rows_v.at[b], acc_sh.at[se_v.at[0, 1]], ssems[b]).wait()
        plsc.subcore_barrier()

        @pl.when(sid == 0)
        def _flush():
            pltpu.sync_copy(acc_sh, out_hbm.at[cid])

    return functools.partial(
        pl.kernel,
        out_type=jax.ShapeDtypeStruct((NC, NPAD, D), jnp.float32),
        mesh=_mesh,
        compiler_params=_sc_params,
        scratch_types=[
            pltpu.VMEM((K1, 2, CHUNK), jnp.int32),
            pltpu.VMEM((NBUF, CHUNK, D), jnp.float32),
            pltpu.VMEM_SHARED((NPAD, D), jnp.float32),
            pltpu.VMEM_SHARED((NPAD, D), jnp.float32),
        ] + [pltpu.SemaphoreType.DMA] * (2 * NBUF),
    )(body)


_scatter64 = _make_scatter(64)
_scatter48 = _make_scatter(48)


def _dis(da_ref, db_ref):
    deg = da_ref[0, :, 0:1] + db_ref[0, :, 0:1] + 1.0
    return lax.rsqrt(deg)


_BLK = NPAD // 4
_BLK3 = N // 5


def _dp_specs():
    return [
        pl.BlockSpec((1, _BLK, DEGW), lambda g: (0, g, 0)),
        pl.BlockSpec((1, _BLK, DEGW), lambda g: (1, g, 0)),
    ]


def _tc1_body(x_ref, w_ref, da_ref, db_ref, y_ref):
    g = pl.program_id(0)
    dis = _dis(da_ref, db_ref)
    xw = jnp.dot(x_ref[...], w_ref[...], preferred_element_type=jnp.float32)
    rows = g * _BLK + lax.broadcasted_iota(jnp.int32, (_BLK, 1), 0)
    y_ref[...] = jnp.where(rows < N, xw * dis, 0.0)


_tc1 = pl.pallas_call(
    _tc1_body,
    grid=(4,),
    in_specs=[
        pl.BlockSpec((_BLK, 128), lambda g: (g, 0)),
        pl.BlockSpec((128, 64), lambda g: (0, 0)),
    ] + _dp_specs(),
    out_specs=pl.BlockSpec((_BLK, 64), lambda g: (g, 0)),
    out_shape=jax.ShapeDtypeStruct((NPAD, 64), jnp.float32),
)


def _tc2_body(z_ref, z_ref2, da_ref, db_ref, w_ref, b1_ref, y2_ref):
    g = pl.program_id(0)
    dis = _dis(da_ref, db_ref)
    pre = (z_ref[0] + z_ref2[0]) * dis + b1_ref[...]
    h = jnp.maximum(pre, 0.0)
    rows = g * _BLK + lax.broadcasted_iota(jnp.int32, (_BLK, 1), 0)
    h = jnp.where(rows < N, h, 0.0)
    y2_ref[...] = jnp.dot(h, w_ref[...], preferred_element_type=jnp.float32) * dis


_tc2 = pl.pallas_call(
    _tc2_body,
    grid=(4,),
    in_specs=[
        pl.BlockSpec((1, _BLK, 64), lambda g: (0, g, 0)),
        pl.BlockSpec((1, _BLK, 64), lambda g: (1, g, 0)),
    ] + _dp_specs() + [
        pl.BlockSpec((64, 48), lambda g: (0, 0)),
        pl.BlockSpec((1, 64), lambda g: (0, 0)),
    ],
    out_specs=pl.BlockSpec((_BLK, 48), lambda g: (g, 0)),
    out_shape=jax.ShapeDtypeStruct((NPAD, 48), jnp.float32),
)


def _tc3_body(z_ref, z_ref2, da_ref, db_ref, b2_ref, out_ref):
    dis = _dis(da_ref, db_ref)
    o = (z_ref[0] + z_ref2[0]) * dis + b2_ref[...]
    cols = lax.broadcasted_iota(jnp.int32, (_BLK3, 48), 1)
    valid = cols < 40
    m = jnp.max(jnp.where(valid, o, -jnp.inf), axis=1, keepdims=True)
    e = jnp.where(valid, jnp.exp(o - m), 0.0)
    s = jnp.sum(e, axis=1, keepdims=True)
    ls = o - m - jnp.log(s)
    out_ref[...] = ls[:, :40]


def _dp_specs3():
    return [
        pl.BlockSpec((1, _BLK3, DEGW), lambda g: (0, g, 0)),
        pl.BlockSpec((1, _BLK3, DEGW), lambda g: (1, g, 0)),
    ]


_tc3 = pl.pallas_call(
    _tc3_body,
    grid=(5,),
    in_specs=[
        pl.BlockSpec((1, _BLK3, 48), lambda g: (0, g, 0)),
        pl.BlockSpec((1, _BLK3, 48), lambda g: (1, g, 0)),
    ] + _dp_specs3() + [
        pl.BlockSpec((1, 48), lambda g: (0, 0)),
    ],
    out_specs=pl.BlockSpec((_BLK3, 40), lambda g: (g, 0)),
    out_shape=jax.ShapeDtypeStruct((N, 40), jnp.float32),
)


def kernel(x, edge_index, W1, b1, W2, b2):
    se0 = edge_index.astype(jnp.int32).reshape(2, NCHUNK, CHUNK).transpose(1, 0, 2)
    se_pad = jnp.full((TCHUNK - NCHUNK, 2, CHUNK), N, dtype=jnp.int32)
    se = jnp.concatenate([se0, se_pad], axis=0)

    w2_p = jnp.pad(W2, ((0, 0), (0, 8)))
    b1_r = b1.reshape(1, 64)
    b2_r = jnp.pad(b2, (0, 8)).reshape(1, 48)
    zeros8 = jnp.zeros((NPAD, DEGW), jnp.float32)
    zeros64 = jnp.zeros((NPAD, 64), jnp.float32)
    zeros48 = jnp.zeros((NPAD, 48), jnp.float32)
    ones8 = jnp.ones((CHUNK, DEGW), jnp.float32)

    dp = _deg_kernel(se, zeros8, ones8)
    y1 = _tc1(x, W1, dp, dp)
    z1 = _scatter64(se, y1, zeros64)
    y2 = _tc2(z1, z1, dp, dp, w2_p, b1_r)
    z2 = _scatter48(se, y2, zeros48)
    out = _tc3(z2, z2, dp, dp, b2_r)
    return out


# speedup vs baseline: 1.0559x; 1.0559x over previous
import functools

import jax
import jax.numpy as jnp
from jax import lax
from jax.experimental import pallas as pl
from jax.experimental.pallas import tpu as pltpu
from jax.experimental.pallas import tpu_sc as plsc

N = 10000
E = 320000
NPAD = 10240
DEGW = 4
NC, NS = 2, 16
NW = NC * NS
CHUNK = 128
NBUF = 2
LAG = 1
K1 = 80
NCHUNK = E // CHUNK
TCHUNK = K1 * NW

_mesh = plsc.VectorSubcoreMesh(
    core_axis_name="c", subcore_axis_name="s", num_cores=NC, num_subcores=NS)
_sc_params = pltpu.CompilerParams(use_tc_tiling_on_sc=False)


def _deg_body(se_hbm, zeros_hbm, ones_hbm, out_hbm, idx_v, ones_v, acc_sh, sem):
    cid = lax.axis_index("c")
    sid = lax.axis_index("s")
    wid = cid * NS + sid
    pltpu.sync_copy(se_hbm.at[pl.ds(wid * K1, K1)], idx_v)
    pltpu.sync_copy(ones_hbm, ones_v)

    @pl.when(sid == 0)
    def _zero():
        pltpu.sync_copy(zeros_hbm, acc_sh)

    plsc.subcore_barrier()

    def body(j, carry):
        pltpu.async_copy(ones_v, acc_sh.at[idx_v.at[j, 1]], sem, add=True)
        return carry

    lax.fori_loop(0, K1, body, 0)

    def drain(j, carry):
        pltpu.make_async_copy(ones_v, acc_sh.at[idx_v.at[0, 1]], sem).wait()
        return carry

    lax.fori_loop(0, K1, drain, 0)
    plsc.subcore_barrier()

    @pl.when(sid == 0)
    def _flush():
        pltpu.sync_copy(acc_sh, out_hbm.at[cid])


_deg_kernel = functools.partial(
    pl.kernel,
    out_type=jax.ShapeDtypeStruct((NC, NPAD, DEGW), jnp.float32),
    mesh=_mesh,
    compiler_params=_sc_params,
    scratch_types=[
        pltpu.VMEM((K1, 2, CHUNK), jnp.int32),
        pltpu.VMEM((CHUNK, DEGW), jnp.float32),
        pltpu.VMEM_SHARED((NPAD, DEGW), jnp.float32),
        pltpu.SemaphoreType.DMA,
    ],
)(_deg_body)


def _make_scatter(D):
    def body(se_hbm, y_hbm, zeros_hbm, out_hbm,
             se_v, rows_v, acc_sh, y_sh, *sems):
        gsems, ssems = sems[:NBUF], sems[NBUF:]
        cid = lax.axis_index("c")
        sid = lax.axis_index("s")
        wid = cid * NS + sid
        pltpu.sync_copy(se_hbm.at[pl.ds(wid * K1, K1)], se_v)

        @pl.when(jnp.logical_and(sid == 0, cid == 0))
        def _init0():
            pltpu.sync_copy(y_hbm, acc_sh)

        @pl.when(jnp.logical_and(sid == 0, cid == 1))
        def _init1():
            pltpu.sync_copy(zeros_hbm, acc_sh)

        @pl.when(sid == 1)
        def _stage():
            pltpu.sync_copy(y_hbm, y_sh)

        plsc.subcore_barrier()
        for g in range(LAG):
            pltpu.async_copy(y_sh.at[se_v.at[g, 0]], rows_v.at[g], gsems[g])

        def step(g, carry):
            for b in range(NBUF):
                j = g * NBUF + b
                pltpu.make_async_copy(
                    y_sh.at[se_v.at[j, 0]], rows_v.at[b], gsems[b]).wait()
                pltpu.async_copy(
                    rows_v.at[b], acc_sh.at[se_v.at[j, 1]], ssems[b], add=True)
                b2 = (b + LAG) % NBUF

                @pl.when(j + LAG < K1)
                def _refill():
                    @pl.when(j >= NBUF - LAG)
                    def _wait_prev_scatter():
                        pltpu.make_async_copy(
                            rows_v.at[b2], acc_sh.at[se_v.at[0, 1]],
                            ssems[b2]).wait()

                    pltpu.async_copy(
                        y_sh.at[se_v.at[j + LAG, 0]], rows_v.at[b2], gsems[b2])
            return carry

        lax.fori_loop(0, K1 // NBUF, step, 0)
        for b in range(NBUF):
            pltpu.make_async_copy(
                rows_v.at[b], acc_sh.at[se_v.at[0, 1]], ssems[b]).wait()
        plsc.subcore_barrier()

        @pl.when(sid == 0)
        def _flush():
            pltpu.sync_copy(acc_sh, out_hbm.at[cid])

    return functools.partial(
        pl.kernel,
        out_type=jax.ShapeDtypeStruct((NC, NPAD, D), jnp.float32),
        mesh=_mesh,
        compiler_params=_sc_params,
        scratch_types=[
            pltpu.VMEM((K1, 2, CHUNK), jnp.int32),
            pltpu.VMEM((NBUF, CHUNK, D), jnp.float32),
            pltpu.VMEM_SHARED((NPAD, D), jnp.float32),
            pltpu.VMEM_SHARED((NPAD, D), jnp.float32),
        ] + [pltpu.SemaphoreType.DMA] * (2 * NBUF),
    )(body)


_scatter64 = _make_scatter(64)
_scatter40 = _make_scatter(40)


def _dis(da_ref, db_ref):
    deg = da_ref[0, :, 0:1] + db_ref[0, :, 0:1] + 1.0
    return lax.rsqrt(deg)


_BLK = NPAD // 4
_BLK3 = N // 5


def _dp_specs():
    return [
        pl.BlockSpec((1, _BLK, DEGW), lambda g: (0, g, 0)),
        pl.BlockSpec((1, _BLK, DEGW), lambda g: (1, g, 0)),
    ]


def _tc1_body(x_ref, w_ref, da_ref, db_ref, y_ref):
    g = pl.program_id(0)
    dis = _dis(da_ref, db_ref)
    xw = jnp.dot(x_ref[...], w_ref[...], preferred_element_type=jnp.float32)
    rows = g * _BLK + lax.broadcasted_iota(jnp.int32, (_BLK, 1), 0)
    y_ref[...] = jnp.where(rows < N, xw * dis, 0.0)


_tc1 = pl.pallas_call(
    _tc1_body,
    grid=(4,),
    in_specs=[
        pl.BlockSpec((_BLK, 128), lambda g: (g, 0)),
        pl.BlockSpec((128, 64), lambda g: (0, 0)),
    ] + _dp_specs(),
    out_specs=pl.BlockSpec((_BLK, 64), lambda g: (g, 0)),
    out_shape=jax.ShapeDtypeStruct((NPAD, 64), jnp.float32),
)


def _tc2_body(z_ref, z_ref2, da_ref, db_ref, w_ref, b1_ref, y2_ref):
    g = pl.program_id(0)
    dis = _dis(da_ref, db_ref)
    pre = (z_ref[0] + z_ref2[0]) * dis + b1_ref[...]
    h = jnp.maximum(pre, 0.0)
    rows = g * _BLK + lax.broadcasted_iota(jnp.int32, (_BLK, 1), 0)
    h = jnp.where(rows < N, h, 0.0)
    y2_ref[...] = jnp.dot(h, w_ref[...], preferred_element_type=jnp.float32) * dis


_tc2 = pl.pallas_call(
    _tc2_body,
    grid=(4,),
    in_specs=[
        pl.BlockSpec((1, _BLK, 64), lambda g: (0, g, 0)),
        pl.BlockSpec((1, _BLK, 64), lambda g: (1, g, 0)),
    ] + _dp_specs() + [
        pl.BlockSpec((64, 40), lambda g: (0, 0)),
        pl.BlockSpec((1, 64), lambda g: (0, 0)),
    ],
    out_specs=pl.BlockSpec((_BLK, 40), lambda g: (g, 0)),
    out_shape=jax.ShapeDtypeStruct((NPAD, 40), jnp.float32),
)


def _tc3_body(z_ref, z_ref2, da_ref, db_ref, b2_ref, out_ref):
    dis = _dis(da_ref, db_ref)
    o = (z_ref[0] + z_ref2[0]) * dis + b2_ref[...]
    m = jnp.max(o, axis=1, keepdims=True)
    s = jnp.sum(jnp.exp(o - m), axis=1, keepdims=True)
    out_ref[...] = o - m - jnp.log(s)


def _dp_specs3():
    return [
        pl.BlockSpec((1, _BLK3, DEGW), lambda g: (0, g, 0)),
        pl.BlockSpec((1, _BLK3, DEGW), lambda g: (1, g, 0)),
    ]


_tc3 = pl.pallas_call(
    _tc3_body,
    grid=(5,),
    in_specs=[
        pl.BlockSpec((1, _BLK3, 40), lambda g: (0, g, 0)),
        pl.BlockSpec((1, _BLK3, 40), lambda g: (1, g, 0)),
    ] + _dp_specs3() + [
        pl.BlockSpec((1, 40), lambda g: (0, 0)),
    ],
    out_specs=pl.BlockSpec((_BLK3, 40), lambda g: (g, 0)),
    out_shape=jax.ShapeDtypeStruct((N, 40), jnp.float32),
)


def kernel(x, edge_index, W1, b1, W2, b2):
    se0 = edge_index.astype(jnp.int32).reshape(2, NCHUNK, CHUNK).transpose(1, 0, 2)
    se_pad = jnp.full((TCHUNK - NCHUNK, 2, CHUNK), N, dtype=jnp.int32)
    se = jnp.concatenate([se0, se_pad], axis=0)

    b1_r = b1.reshape(1, 64)
    b2_r = b2.reshape(1, 40)
    zeros8 = jnp.zeros((NPAD, DEGW), jnp.float32)
    zeros64 = jnp.zeros((NPAD, 64), jnp.float32)
    zeros40 = jnp.zeros((NPAD, 40), jnp.float32)
    ones8 = jnp.ones((CHUNK, DEGW), jnp.float32)

    dp = _deg_kernel(se, zeros8, ones8)
    y1 = _tc1(x, W1, dp, dp)
    z1 = _scatter64(se, y1, zeros64)
    y2 = _tc2(z1, z1, dp, dp, W2, b1_r)
    z2 = _scatter40(se, y2, zeros40)
    out = _tc3(z2, z2, dp, dp, b2_r)
    return out


# speedup vs baseline: 1.0656x; 1.0092x over previous
import functools

import jax
import jax.numpy as jnp
from jax import lax
from jax.experimental import pallas as pl
from jax.experimental.pallas import tpu as pltpu
from jax.experimental.pallas import tpu_sc as plsc

N = 10000
E = 320000
NPAD = 10240
DEGW = 8
NC, NS = 2, 16
NW = NC * NS
CHUNK = 128
NBUF = 2
LAG = 1
K1 = 80
NCHUNK = E // CHUNK
TCHUNK = K1 * NW

_mesh = plsc.VectorSubcoreMesh(
    core_axis_name="c", subcore_axis_name="s", num_cores=NC, num_subcores=NS)
_sc_params = pltpu.CompilerParams(use_tc_tiling_on_sc=False)


def _deg_body(se_hbm, zeros_hbm, ones_hbm, out_hbm, idx_v, ones_v, acc_sh, sem):
    cid = lax.axis_index("c")
    sid = lax.axis_index("s")
    wid = cid * NS + sid
    pltpu.sync_copy(se_hbm.at[pl.ds(wid * K1, K1)], idx_v)
    pltpu.sync_copy(ones_hbm, ones_v)

    @pl.when(sid == 0)
    def _zero():
        pltpu.sync_copy(zeros_hbm, acc_sh)

    plsc.subcore_barrier()

    def body(j, carry):
        pltpu.async_copy(ones_v, acc_sh.at[idx_v.at[j, 1]], sem, add=True)
        return carry

    lax.fori_loop(0, K1, body, 0)

    def drain(j, carry):
        pltpu.make_async_copy(ones_v, acc_sh.at[idx_v.at[0, 1]], sem).wait()
        return carry

    lax.fori_loop(0, K1, drain, 0)
    plsc.subcore_barrier()

    @pl.when(sid == 0)
    def _flush():
        pltpu.sync_copy(acc_sh, out_hbm.at[cid])


_deg_kernel = functools.partial(
    pl.kernel,
    out_type=jax.ShapeDtypeStruct((NC, NPAD, DEGW), jnp.float32),
    mesh=_mesh,
    compiler_params=_sc_params,
    scratch_types=[
        pltpu.VMEM((K1, 2, CHUNK), jnp.int32),
        pltpu.VMEM((CHUNK, DEGW), jnp.float32),
        pltpu.VMEM_SHARED((NPAD, DEGW), jnp.float32),
        pltpu.SemaphoreType.DMA,
    ],
)(_deg_body)


def _make_scatter(D):
    def body(se_hbm, y_hbm, zeros_hbm, out_hbm,
             se_v, rows_v, acc_sh, y_sh, *sems):
        gsems, ssems = sems[:NBUF], sems[NBUF:]
        cid = lax.axis_index("c")
        sid = lax.axis_index("s")
        wid = cid * NS + sid
        pltpu.sync_copy(se_hbm.at[pl.ds(wid * K1, K1)], se_v)

        @pl.when(jnp.logical_and(sid == 0, cid == 0))
        def _init0():
            pltpu.sync_copy(y_hbm, acc_sh)

        @pl.when(jnp.logical_and(sid == 0, cid == 1))
        def _init1():
            pltpu.sync_copy(zeros_hbm, acc_sh)

        @pl.when(sid == 1)
        def _stage():
            pltpu.sync_copy(y_hbm, y_sh)

        plsc.subcore_barrier()
        for g in range(LAG):
            pltpu.async_copy(y_sh.at[se_v.at[g, 0]], rows_v.at[g], gsems[g])

        def step(g, carry):
            for b in range(NBUF):
                j = g * NBUF + b
                pltpu.make_async_copy(
                    y_sh.at[se_v.at[j, 0]], rows_v.at[b], gsems[b]).wait()
                pltpu.async_copy(
                    rows_v.at[b], acc_sh.at[se_v.at[j, 1]], ssems[b], add=True)
                b2 = (b + LAG) % NBUF

                @pl.when(j + LAG < K1)
                def _refill():
                    @pl.when(j >= NBUF - LAG)
                    def _wait_prev_scatter():
                        pltpu.make_async_copy(
                            rows_v.at[b2], acc_sh.at[se_v.at[0, 1]],
                            ssems[b2]).wait()

                    pltpu.async_copy(
                        y_sh.at[se_v.at[j + LAG, 0]], rows_v.at[b2], gsems[b2])
            return carry

        lax.fori_loop(0, K1 // NBUF, step, 0)
        for b in range(NBUF):
            pltpu.make_async_copy(
                rows_v.at[b], acc_sh.at[se_v.at[0, 1]], ssems[b]).wait()
        plsc.subcore_barrier()

        @pl.when(sid == 0)
        def _flush():
            pltpu.sync_copy(acc_sh, out_hbm.at[cid])

    return functools.partial(
        pl.kernel,
        out_type=jax.ShapeDtypeStruct((NC, NPAD, D), jnp.float32),
        mesh=_mesh,
        compiler_params=_sc_params,
        scratch_types=[
            pltpu.VMEM((K1, 2, CHUNK), jnp.int32),
            pltpu.VMEM((NBUF, CHUNK, D), jnp.float32),
            pltpu.VMEM_SHARED((NPAD, D), jnp.float32),
            pltpu.VMEM_SHARED((NPAD, D), jnp.float32),
        ] + [pltpu.SemaphoreType.DMA] * (2 * NBUF),
    )(body)


_scatter64 = _make_scatter(64)
_scatter40 = _make_scatter(40)


def _dis(da_ref, db_ref):
    deg = da_ref[0, :, 0:1] + db_ref[0, :, 0:1] + 1.0
    return lax.rsqrt(deg)


_BLK = NPAD // 4
_BLK3 = N // 5


def _dp_specs():
    return [
        pl.BlockSpec((1, _BLK, DEGW), lambda g: (0, g, 0)),
        pl.BlockSpec((1, _BLK, DEGW), lambda g: (1, g, 0)),
    ]


def _tc1_body(x_ref, w_ref, da_ref, db_ref, y_ref):
    g = pl.program_id(0)
    dis = _dis(da_ref, db_ref)
    xw = jnp.dot(x_ref[...], w_ref[...], preferred_element_type=jnp.float32)
    rows = g * _BLK + lax.broadcasted_iota(jnp.int32, (_BLK, 1), 0)
    y_ref[...] = jnp.where(rows < N, xw * dis, 0.0)


_tc1 = pl.pallas_call(
    _tc1_body,
    grid=(4,),
    in_specs=[
        pl.BlockSpec((_BLK, 128), lambda g: (g, 0)),
        pl.BlockSpec((128, 64), lambda g: (0, 0)),
    ] + _dp_specs(),
    out_specs=pl.BlockSpec((_BLK, 64), lambda g: (g, 0)),
    out_shape=jax.ShapeDtypeStruct((NPAD, 64), jnp.float32),
)


def _tc2_body(z_ref, z_ref2, da_ref, db_ref, w_ref, b1_ref, y2_ref):
    g = pl.program_id(0)
    dis = _dis(da_ref, db_ref)
    pre = (z_ref[0] + z_ref2[0]) * dis + b1_ref[...]
    h = jnp.maximum(pre, 0.0)
    rows = g * _BLK + lax.broadcasted_iota(jnp.int32, (_BLK, 1), 0)
    h = jnp.where(rows < N, h, 0.0)
    y2_ref[...] = jnp.dot(h, w_ref[...], preferred_element_type=jnp.float32) * dis


_tc2 = pl.pallas_call(
    _tc2_body,
    grid=(4,),
    in_specs=[
        pl.BlockSpec((1, _BLK, 64), lambda g: (0, g, 0)),
        pl.BlockSpec((1, _BLK, 64), lambda g: (1, g, 0)),
    ] + _dp_specs() + [
        pl.BlockSpec((64, 40), lambda g: (0, 0)),
        pl.BlockSpec((1, 64), lambda g: (0, 0)),
    ],
    out_specs=pl.BlockSpec((_BLK, 40), lambda g: (g, 0)),
    out_shape=jax.ShapeDtypeStruct((NPAD, 40), jnp.float32),
)


def _tc3_body(z_ref, z_ref2, da_ref, db_ref, b2_ref, out_ref):
    dis = _dis(da_ref, db_ref)
    o = (z_ref[0] + z_ref2[0]) * dis + b2_ref[...]
    m = jnp.max(o, axis=1, keepdims=True)
    s = jnp.sum(jnp.exp(o - m), axis=1, keepdims=True)
    out_ref[...] = o - m - jnp.log(s)


def _dp_specs3():
    return [
        pl.BlockSpec((1, _BLK3, DEGW), lambda g: (0, g, 0)),
        pl.BlockSpec((1, _BLK3, DEGW), lambda g: (1, g, 0)),
    ]


_tc3 = pl.pallas_call(
    _tc3_body,
    grid=(5,),
    in_specs=[
        pl.BlockSpec((1, _BLK3, 40), lambda g: (0, g, 0)),
        pl.BlockSpec((1, _BLK3, 40), lambda g: (1, g, 0)),
    ] + _dp_specs3() + [
        pl.BlockSpec((1, 40), lambda g: (0, 0)),
    ],
    out_specs=pl.BlockSpec((_BLK3, 40), lambda g: (g, 0)),
    out_shape=jax.ShapeDtypeStruct((N, 40), jnp.float32),
)


def kernel(x, edge_index, W1, b1, W2, b2):
    se0 = edge_index.astype(jnp.int32).reshape(2, NCHUNK, CHUNK).transpose(1, 0, 2)
    se_pad = jnp.full((TCHUNK - NCHUNK, 2, CHUNK), N, dtype=jnp.int32)
    se = jnp.concatenate([se0, se_pad], axis=0)

    b1_r = b1.reshape(1, 64)
    b2_r = b2.reshape(1, 40)
    zeros8 = jnp.zeros((NPAD, DEGW), jnp.float32)
    zeros64 = jnp.zeros((NPAD, 64), jnp.float32)
    zeros40 = jnp.zeros((NPAD, 40), jnp.float32)
    ones8 = jnp.ones((CHUNK, DEGW), jnp.float32)

    dp = _deg_kernel(se, zeros8, ones8)
    y1 = _tc1(x, W1, dp, dp)
    z1 = _scatter64(se, y1, zeros64)
    y2 = _tc2(z1, z1, dp, dp, W2, b1_r)
    z2 = _scatter40(se, y2, zeros40)
    out = _tc3(z2, z2, dp, dp, b2_r)
    return out


# speedup vs baseline: 1.2331x; 1.1572x over previous
import functools

import jax
import jax.numpy as jnp
from jax import lax
from jax.experimental import pallas as pl
from jax.experimental.pallas import tpu as pltpu
from jax.experimental.pallas import tpu_sc as plsc

N = 10000
E = 320000
NPAD = 10240
DEGW = 8
NC, NS = 2, 16
NW = NC * NS
CHUNK = 128
NBUF = 2
LAG = 1
K1 = 80
NCHUNK = E // CHUNK
TCHUNK = K1 * NW

_mesh = plsc.VectorSubcoreMesh(
    core_axis_name="c", subcore_axis_name="s", num_cores=NC, num_subcores=NS)
_sc_params = pltpu.CompilerParams(use_tc_tiling_on_sc=False)


def _deg_body(se_hbm, zeros_hbm, ones_hbm, out_hbm, idx_v, ones_v, acc_sh, sem):
    cid = lax.axis_index("c")
    sid = lax.axis_index("s")
    wid = cid * NS + sid
    pltpu.sync_copy(se_hbm.at[pl.ds(wid * K1, K1)], idx_v)
    pltpu.sync_copy(ones_hbm, ones_v)

    @pl.when(sid == 0)
    def _zero():
        pltpu.sync_copy(zeros_hbm, acc_sh)

    plsc.subcore_barrier()

    def body(j, carry):
        pltpu.async_copy(ones_v, acc_sh.at[idx_v.at[j, 1]], sem, add=True)
        return carry

    lax.fori_loop(0, K1, body, 0)

    def drain(j, carry):
        pltpu.make_async_copy(ones_v, acc_sh.at[idx_v.at[0, 1]], sem).wait()
        return carry

    lax.fori_loop(0, K1, drain, 0)
    plsc.subcore_barrier()

    @pl.when(sid == 0)
    def _flush():
        pltpu.sync_copy(acc_sh, out_hbm.at[cid])


_deg_kernel = functools.partial(
    pl.kernel,
    out_type=jax.ShapeDtypeStruct((NC, NPAD, DEGW), jnp.float32),
    mesh=_mesh,
    compiler_params=_sc_params,
    scratch_types=[
        pltpu.VMEM((K1, 2, CHUNK), jnp.int32),
        pltpu.VMEM((CHUNK, DEGW), jnp.float32),
        pltpu.VMEM_SHARED((NPAD, DEGW), jnp.float32),
        pltpu.SemaphoreType.DMA,
    ],
)(_deg_body)


def _make_scatter(D, dtype=jnp.float32):
    def body(se_hbm, y_hbm, zeros_hbm, out_hbm,
             se_v, rows_v, acc_sh, y_sh, *sems):
        gsems, ssems = sems[:NBUF], sems[NBUF:]
        cid = lax.axis_index("c")
        sid = lax.axis_index("s")
        wid = cid * NS + sid
        pltpu.sync_copy(se_hbm.at[pl.ds(wid * K1, K1)], se_v)

        @pl.when(jnp.logical_and(sid == 0, cid == 0))
        def _init0():
            pltpu.sync_copy(y_hbm, acc_sh)

        @pl.when(jnp.logical_and(sid == 0, cid == 1))
        def _init1():
            pltpu.sync_copy(zeros_hbm, acc_sh)

        @pl.when(sid == 1)
        def _stage():
            pltpu.sync_copy(y_hbm, y_sh)

        plsc.subcore_barrier()
        for g in range(LAG):
            pltpu.async_copy(y_sh.at[se_v.at[g, 0]], rows_v.at[g], gsems[g])

        def step(g, carry):
            for b in range(NBUF):
                j = g * NBUF + b
                pltpu.make_async_copy(
                    y_sh.at[se_v.at[j, 0]], rows_v.at[b], gsems[b]).wait()
                pltpu.async_copy(
                    rows_v.at[b], acc_sh.at[se_v.at[j, 1]], ssems[b], add=True)
                b2 = (b + LAG) % NBUF

                @pl.when(j + LAG < K1)
                def _refill():
                    @pl.when(j >= NBUF - LAG)
                    def _wait_prev_scatter():
                        pltpu.make_async_copy(
                            rows_v.at[b2], acc_sh.at[se_v.at[0, 1]],
                            ssems[b2]).wait()

                    pltpu.async_copy(
                        y_sh.at[se_v.at[j + LAG, 0]], rows_v.at[b2], gsems[b2])
            return carry

        lax.fori_loop(0, K1 // NBUF, step, 0)
        for b in range(NBUF):
            pltpu.make_async_copy(
                rows_v.at[b], acc_sh.at[se_v.at[0, 1]], ssems[b]).wait()
        plsc.subcore_barrier()

        @pl.when(sid == 0)
        def _flush():
            pltpu.sync_copy(acc_sh, out_hbm.at[cid])

    return functools.partial(
        pl.kernel,
        out_type=jax.ShapeDtypeStruct((NC, NPAD, D), dtype),
        mesh=_mesh,
        compiler_params=_sc_params,
        scratch_types=[
            pltpu.VMEM((K1, 2, CHUNK), jnp.int32),
            pltpu.VMEM((NBUF, CHUNK, D), dtype),
            pltpu.VMEM_SHARED((NPAD, D), dtype),
            pltpu.VMEM_SHARED((NPAD, D), dtype),
        ] + [pltpu.SemaphoreType.DMA] * (2 * NBUF),
    )(body)


_scatter64 = _make_scatter(64, jnp.bfloat16)
_scatter40 = _make_scatter(40)


def _dis(da_ref, db_ref):
    deg = da_ref[0, :, 0:1] + db_ref[0, :, 0:1] + 1.0
    return lax.rsqrt(deg)


_BLK = NPAD // 4
_BLK3 = N // 5


def _dp_specs():
    return [
        pl.BlockSpec((1, _BLK, DEGW), lambda g: (0, g, 0)),
        pl.BlockSpec((1, _BLK, DEGW), lambda g: (1, g, 0)),
    ]


def _tc1_body(x_ref, w_ref, da_ref, db_ref, y_ref):
    g = pl.program_id(0)
    dis = _dis(da_ref, db_ref)
    xw = jnp.dot(x_ref[...], w_ref[...], preferred_element_type=jnp.float32)
    rows = g * _BLK + lax.broadcasted_iota(jnp.int32, (_BLK, 1), 0)
    y_ref[...] = jnp.where(rows < N, xw * dis, 0.0).astype(jnp.bfloat16)


_tc1 = pl.pallas_call(
    _tc1_body,
    grid=(4,),
    in_specs=[
        pl.BlockSpec((_BLK, 128), lambda g: (g, 0)),
        pl.BlockSpec((128, 64), lambda g: (0, 0)),
    ] + _dp_specs(),
    out_specs=pl.BlockSpec((_BLK, 64), lambda g: (g, 0)),
    out_shape=jax.ShapeDtypeStruct((NPAD, 64), jnp.bfloat16),
)


def _tc2_body(z_ref, z_ref2, da_ref, db_ref, w_ref, b1_ref, y2_ref):
    g = pl.program_id(0)
    dis = _dis(da_ref, db_ref)
    z = z_ref[0].astype(jnp.float32) + z_ref2[0].astype(jnp.float32)
    pre = z * dis + b1_ref[...]
    h = jnp.maximum(pre, 0.0)
    rows = g * _BLK + lax.broadcasted_iota(jnp.int32, (_BLK, 1), 0)
    h = jnp.where(rows < N, h, 0.0)
    y2_ref[...] = jnp.dot(h, w_ref[...], preferred_element_type=jnp.float32) * dis


_tc2 = pl.pallas_call(
    _tc2_body,
    grid=(4,),
    in_specs=[
        pl.BlockSpec((1, _BLK, 64), lambda g: (0, g, 0)),
        pl.BlockSpec((1, _BLK, 64), lambda g: (1, g, 0)),
    ] + _dp_specs() + [
        pl.BlockSpec((64, 40), lambda g: (0, 0)),
        pl.BlockSpec((1, 64), lambda g: (0, 0)),
    ],
    out_specs=pl.BlockSpec((_BLK, 40), lambda g: (g, 0)),
    out_shape=jax.ShapeDtypeStruct((NPAD, 40), jnp.float32),
)


def _tc3_body(z_ref, z_ref2, da_ref, db_ref, b2_ref, out_ref):
    dis = _dis(da_ref, db_ref)
    o = (z_ref[0] + z_ref2[0]) * dis + b2_ref[...]
    m = jnp.max(o, axis=1, keepdims=True)
    s = jnp.sum(jnp.exp(o - m), axis=1, keepdims=True)
    out_ref[...] = o - m - jnp.log(s)


def _dp_specs3():
    return [
        pl.BlockSpec((1, _BLK3, DEGW), lambda g: (0, g, 0)),
        pl.BlockSpec((1, _BLK3, DEGW), lambda g: (1, g, 0)),
    ]


_tc3 = pl.pallas_call(
    _tc3_body,
    grid=(5,),
    in_specs=[
        pl.BlockSpec((1, _BLK3, 40), lambda g: (0, g, 0)),
        pl.BlockSpec((1, _BLK3, 40), lambda g: (1, g, 0)),
    ] + _dp_specs3() + [
        pl.BlockSpec((1, 40), lambda g: (0, 0)),
    ],
    out_specs=pl.BlockSpec((_BLK3, 40), lambda g: (g, 0)),
    out_shape=jax.ShapeDtypeStruct((N, 40), jnp.float32),
)


def kernel(x, edge_index, W1, b1, W2, b2):
    se0 = edge_index.astype(jnp.int32).reshape(2, NCHUNK, CHUNK).transpose(1, 0, 2)
    se_pad = jnp.full((TCHUNK - NCHUNK, 2, CHUNK), N, dtype=jnp.int32)
    se = jnp.concatenate([se0, se_pad], axis=0)

    b1_r = b1.reshape(1, 64)
    b2_r = b2.reshape(1, 40)
    zeros8 = jnp.zeros((NPAD, DEGW), jnp.float32)
    zeros64 = jnp.zeros((NPAD, 64), jnp.bfloat16)
    zeros40 = jnp.zeros((NPAD, 40), jnp.float32)
    ones8 = jnp.ones((CHUNK, DEGW), jnp.float32)

    dp = _deg_kernel(se, zeros8, ones8)
    y1 = _tc1(x, W1, dp, dp)
    z1 = _scatter64(se, y1, zeros64)
    y2 = _tc2(z1, z1, dp, dp, W2, b1_r)
    z2 = _scatter40(se, y2, zeros40)
    out = _tc3(z2, z2, dp, dp, b2_r)
    return out
